# Initial kernel scaffold; baseline (speedup 1.0000x reference)
import functools

import jax
import jax.numpy as jnp
from jax import lax
from jax.experimental import pallas as pl
from jax.experimental.pallas import tpu as pltpu
from jax.experimental.pallas import tpu_sc as plsc

N_NODES = 10000
N_EDGES = 320000
N_GRAPHS = 64
IN_DIM = 128
HID_DIM = 64
OUT_DIM = 64
LAYERS = 4
EPS_BN = 1e-5


_NC = 2
_NS = 16
_NW = _NC * _NS
_CHUNK = 128
_NCHUNKS = N_EDGES // _CHUNK
_ROWS_PER_TILE = N_NODES // _NS


def _sc_edge_agg(h, src, dst, zrows):
    mesh = plsc.VectorSubcoreMesh(core_axis_name="c", subcore_axis_name="s")

    @functools.partial(
        pl.kernel,
        mesh=mesh,
        out_type=jax.ShapeDtypeStruct((_NC, N_NODES, HID_DIM), jnp.float32),
        scratch_types=[
            pltpu.VMEM((_CHUNK,), jnp.int32),
            pltpu.VMEM((_CHUNK,), jnp.int32),
            pltpu.VMEM((_CHUNK, HID_DIM), jnp.float32),
            pltpu.VMEM_SHARED((N_NODES, HID_DIM), jnp.float32),
            pltpu.SemaphoreType.DMA,
        ],
    )
    def agg_kernel(h_hbm, src_hbm, dst_hbm, z_hbm, out_hbm,
                   src_v, dst_v, rows_v, acc_sh, sem):
        c = lax.axis_index("c")
        s = lax.axis_index("s")
        w = s * _NC + c

        r0 = s * _ROWS_PER_TILE
        pltpu.sync_copy(z_hbm.at[pl.ds(0, _ROWS_PER_TILE)],
                        acc_sh.at[pl.ds(r0, _ROWS_PER_TILE)])
        plsc.subcore_barrier()

        nch = (_NCHUNKS - 1 - w) // _NW + 1

        def body(i, carry):
            base = (w + i * _NW) * _CHUNK
            pltpu.sync_copy(src_hbm.at[pl.ds(base, _CHUNK)], src_v)
            pltpu.sync_copy(dst_hbm.at[pl.ds(base, _CHUNK)], dst_v)
            pltpu.async_copy(h_hbm.at[src_v], rows_v, sem).wait()
            pltpu.sync_copy(rows_v, acc_sh.at[dst_v], add=True)
            return carry

        lax.fori_loop(0, nch, body, 0)
        plsc.subcore_barrier()

        pltpu.sync_copy(acc_sh.at[pl.ds(r0, _ROWS_PER_TILE)],
                        out_hbm.at[c, pl.ds(r0, _ROWS_PER_TILE)])

    return agg_kernel(h, src, dst, zrows)


_PREC = lax.Precision.HIGHEST


def _bn_relu(y, g, b):
    m = jnp.mean(y, axis=0, keepdims=True)
    yc = y - m
    v = jnp.mean(yc * yc, axis=0, keepdims=True)
    return jnp.maximum(yc * lax.rsqrt(v + EPS_BN) * g + b, 0.0)


def _pool_project(h, bt_row, wlt, bl):
    oneh = (lax.broadcasted_iota(jnp.int32, (N_GRAPHS, N_NODES), 0)
            == bt_row).astype(jnp.float32)
    sums = jnp.dot(oneh, h, precision=_PREC)
    cnt = jnp.sum(oneh, axis=1, keepdims=True)
    pooled = sums / jnp.maximum(cnt, 1.0)
    return jnp.dot(pooled, wlt, precision=_PREC) + bl


def _first_body(x_ref, bt_ref, w1t_ref, b1_ref, g1_ref, be1_ref,
                w2t_ref, b2_ref, g2_ref, be2_ref, wlt_ref, bl_ref,
                h_ref, outp_ref):
    y = jnp.dot(x_ref[...], w1t_ref[...], precision=_PREC) + b1_ref[...]
    y = _bn_relu(y, g1_ref[...], be1_ref[...])
    y = jnp.dot(y, w2t_ref[...], precision=_PREC) + b2_ref[...]
    h = _bn_relu(y, g2_ref[...], be2_ref[...])
    h_ref[...] = h
    outp_ref[...] = _pool_project(h, bt_ref[...], wlt_ref[...], bl_ref[...])


def _layer_body(h_in_ref, agg_ref, eps_ref, bt_ref, w1t_ref, b1_ref, g1_ref,
                be1_ref, w2t_ref, b2_ref, g2_ref, be2_ref, wlt_ref, bl_ref,
                h_ref, outp_ref):
    u = (h_in_ref[...] * (1.0 + eps_ref[...])
         + agg_ref[0] + agg_ref[1])
    y = jnp.dot(u, w1t_ref[...], precision=_PREC) + b1_ref[...]
    y = _bn_relu(y, g1_ref[...], be1_ref[...])
    y = jnp.dot(y, w2t_ref[...], precision=_PREC) + b2_ref[...]
    h = _bn_relu(y, g2_ref[...], be2_ref[...])
    h_ref[...] = h
    outp_ref[...] = _pool_project(h, bt_ref[...], wlt_ref[...], bl_ref[...])


_DENSE_OUT = (
    jax.ShapeDtypeStruct((N_NODES, HID_DIM), jnp.float32),
    jax.ShapeDtypeStruct((N_GRAPHS, OUT_DIM), jnp.float32),
)

_first_call = pl.pallas_call(_first_body, out_shape=_DENSE_OUT)
_layer_call = pl.pallas_call(_layer_body, out_shape=_DENSE_OUT)


def _mlp_args(p):
    return (p["W1"].T, p["b1"].reshape(1, -1), p["g1"].reshape(1, -1),
            p["be1"].reshape(1, -1), p["W2"].T, p["b2"].reshape(1, -1),
            p["g2"].reshape(1, -1), p["be2"].reshape(1, -1))


def kernel(x, edge_index, batch, params):
    src = edge_index[0]
    dst = edge_index[1]
    bt_row = batch.reshape(1, N_NODES)
    zrows = jnp.zeros((_ROWS_PER_TILE, HID_DIM), jnp.float32)

    lin = params["lin"]
    h, out = _first_call(
        x, bt_row, *_mlp_args(params["first_h"]),
        lin[0]["W"].T, lin[0]["b"].reshape(1, -1))

    for layer in range(1, LAYERS):
        agg = _sc_edge_agg(h, src, dst, zrows)
        eps = params["eps"][layer - 1].reshape(1, 1)
        h, outp = _layer_call(
            h, agg, eps, bt_row, *_mlp_args(params["nns"][layer - 1]),
            lin[layer]["W"].T, lin[layer]["b"].reshape(1, -1))
        out = out + outp
    return out


# speedup vs baseline: 5.5461x; 5.5461x over previous
import functools

import jax
import jax.numpy as jnp
from jax import lax
from jax.experimental import pallas as pl
from jax.experimental.pallas import tpu as pltpu
from jax.experimental.pallas import tpu_sc as plsc

N_NODES = 10000
N_EDGES = 320000
N_GRAPHS = 64
IN_DIM = 128
HID_DIM = 64
OUT_DIM = 64
LAYERS = 4
EPS_BN = 1e-5


_NC = 2
_NS = 16
_NW = _NC * _NS
_CHUNK = 128
_NCHUNKS = N_EDGES // _CHUNK
_ROWS_PER_TILE = 632
_N_PAD = _ROWS_PER_TILE * _NS


def _sc_edge_agg(h, src, dst, zrows):
    mesh = plsc.VectorSubcoreMesh(core_axis_name="c", subcore_axis_name="s")

    @functools.partial(
        pl.kernel,
        mesh=mesh,
        out_type=jax.ShapeDtypeStruct((_NC, _N_PAD, HID_DIM), jnp.float32),
        scratch_types=[
            pltpu.VMEM((_CHUNK,), jnp.int32),
            pltpu.VMEM((_CHUNK,), jnp.int32),
            pltpu.VMEM((_CHUNK, HID_DIM), jnp.float32),
            pltpu.VMEM_SHARED((_N_PAD, HID_DIM), jnp.float32),
            pltpu.SemaphoreType.DMA,
        ],
        compiler_params=pltpu.CompilerParams(use_tc_tiling_on_sc=False),
    )
    def agg_kernel(h_hbm, src_hbm, dst_hbm, z_hbm, out_hbm,
                   src_v, dst_v, rows_v, acc_sh, sem):
        c = lax.axis_index("c")
        s = lax.axis_index("s")
        w = s * _NC + c

        r0 = s * _ROWS_PER_TILE
        pltpu.sync_copy(z_hbm.at[pl.ds(0, _ROWS_PER_TILE)],
                        acc_sh.at[pl.ds(r0, _ROWS_PER_TILE)])
        plsc.subcore_barrier()

        nch = (_NCHUNKS - 1 - w) // _NW + 1

        def body(i, carry):
            base = (w + i * _NW) * _CHUNK
            pltpu.sync_copy(src_hbm.at[pl.ds(base, _CHUNK)], src_v)
            pltpu.sync_copy(dst_hbm.at[pl.ds(base, _CHUNK)], dst_v)
            pltpu.async_copy(h_hbm.at[src_v], rows_v, sem).wait()
            pltpu.sync_copy(rows_v, acc_sh.at[dst_v], add=True)
            return carry

        lax.fori_loop(0, nch, body, 0)
        plsc.subcore_barrier()

        pltpu.sync_copy(acc_sh.at[pl.ds(r0, _ROWS_PER_TILE)],
                        out_hbm.at[c, pl.ds(r0, _ROWS_PER_TILE)])

    return agg_kernel(h, src, dst, zrows)


_PREC = lax.Precision.HIGHEST


def _bn_relu(y, g, b):
    m = jnp.mean(y, axis=0, keepdims=True)
    yc = y - m
    v = jnp.mean(yc * yc, axis=0, keepdims=True)
    return jnp.maximum(yc * lax.rsqrt(v + EPS_BN) * g + b, 0.0)


def _pool_project(h, bt_row, wlt, bl):
    oneh = (lax.broadcasted_iota(jnp.int32, (N_GRAPHS, N_NODES), 0)
            == bt_row).astype(jnp.float32)
    sums = jnp.dot(oneh, h, precision=_PREC)
    cnt = jnp.sum(oneh, axis=1, keepdims=True)
    pooled = sums / jnp.maximum(cnt, 1.0)
    return jnp.dot(pooled, wlt, precision=_PREC) + bl


def _first_body(x_ref, bt_ref, w1t_ref, b1_ref, g1_ref, be1_ref,
                w2t_ref, b2_ref, g2_ref, be2_ref, wlt_ref, bl_ref,
                h_ref, outp_ref):
    y = jnp.dot(x_ref[...], w1t_ref[...], precision=_PREC) + b1_ref[...]
    y = _bn_relu(y, g1_ref[...], be1_ref[...])
    y = jnp.dot(y, w2t_ref[...], precision=_PREC) + b2_ref[...]
    h = _bn_relu(y, g2_ref[...], be2_ref[...])
    h_ref[...] = h
    outp_ref[...] = _pool_project(h, bt_ref[...], wlt_ref[...], bl_ref[...])


def _layer_body(h_in_ref, agg_ref, eps_ref, bt_ref, w1t_ref, b1_ref, g1_ref,
                be1_ref, w2t_ref, b2_ref, g2_ref, be2_ref, wlt_ref, bl_ref,
                h_ref, outp_ref):
    u = (h_in_ref[...] * (1.0 + eps_ref[...])
         + agg_ref[0, :N_NODES, :] + agg_ref[1, :N_NODES, :])
    y = jnp.dot(u, w1t_ref[...], precision=_PREC) + b1_ref[...]
    y = _bn_relu(y, g1_ref[...], be1_ref[...])
    y = jnp.dot(y, w2t_ref[...], precision=_PREC) + b2_ref[...]
    h = _bn_relu(y, g2_ref[...], be2_ref[...])
    h_ref[...] = h
    outp_ref[...] = _pool_project(h, bt_ref[...], wlt_ref[...], bl_ref[...])


_DENSE_OUT = (
    jax.ShapeDtypeStruct((N_NODES, HID_DIM), jnp.float32),
    jax.ShapeDtypeStruct((N_GRAPHS, OUT_DIM), jnp.float32),
)

_first_call = pl.pallas_call(_first_body, out_shape=_DENSE_OUT)
_layer_call = pl.pallas_call(_layer_body, out_shape=_DENSE_OUT)


def _mlp_args(p):
    return (p["W1"].T, p["b1"].reshape(1, -1), p["g1"].reshape(1, -1),
            p["be1"].reshape(1, -1), p["W2"].T, p["b2"].reshape(1, -1),
            p["g2"].reshape(1, -1), p["be2"].reshape(1, -1))


def kernel(x, edge_index, batch, params):
    src = edge_index[0]
    dst = edge_index[1]
    bt_row = batch.reshape(1, N_NODES)
    zrows = jnp.zeros((_ROWS_PER_TILE, HID_DIM), jnp.float32)

    lin = params["lin"]
    h, out = _first_call(
        x, bt_row, *_mlp_args(params["first_h"]),
        lin[0]["W"].T, lin[0]["b"].reshape(1, -1))

    for layer in range(1, LAYERS):
        agg = _sc_edge_agg(h, src, dst, zrows)
        eps = params["eps"][layer - 1].reshape(1, 1)
        h, outp = _layer_call(
            h, agg, eps, bt_row, *_mlp_args(params["nns"][layer - 1]),
            lin[layer]["W"].T, lin[layer]["b"].reshape(1, -1))
        out = out + outp
    return out


# speedup vs baseline: 8.8362x; 1.5932x over previous
import functools

import jax
import jax.numpy as jnp
from jax import lax
from jax.experimental import pallas as pl
from jax.experimental.pallas import tpu as pltpu
from jax.experimental.pallas import tpu_sc as plsc

N_NODES = 10000
N_EDGES = 320000
N_GRAPHS = 64
IN_DIM = 128
HID_DIM = 64
OUT_DIM = 64
LAYERS = 4
EPS_BN = 1e-5


_NC = 2
_NS = 16
_NW = _NC * _NS
_CHUNK = 125
_NCHUNKS = N_EDGES // _CHUNK
_CH_PER_W = _NCHUNKS // _NW
_ROWS_PER_TILE = 632
_N_PAD = _ROWS_PER_TILE * _NS


def _sc_edge_agg(h, src2d, dst2d, zrows):
    mesh = plsc.VectorSubcoreMesh(core_axis_name="c", subcore_axis_name="s")

    @functools.partial(
        pl.kernel,
        mesh=mesh,
        out_type=jax.ShapeDtypeStruct((_NC, _N_PAD, HID_DIM), jnp.float32),
        scratch_types=[
            pltpu.VMEM((_CH_PER_W, _CHUNK), jnp.int32),
            pltpu.VMEM((_CH_PER_W, _CHUNK), jnp.int32),
            pltpu.VMEM((_CHUNK, HID_DIM), jnp.float32),
            pltpu.VMEM((_CHUNK, HID_DIM), jnp.float32),
            pltpu.VMEM_SHARED((_N_PAD, HID_DIM), jnp.float32),
            pltpu.SemaphoreType.DMA,
        ],
        compiler_params=pltpu.CompilerParams(use_tc_tiling_on_sc=False),
    )
    def agg_kernel(h_hbm, src_hbm, dst_hbm, z_hbm, out_hbm,
                   src_v, dst_v, rows0, rows1, acc_sh, gsem):
        c = lax.axis_index("c")
        s = lax.axis_index("s")
        w = s * _NC + c

        r0 = s * _ROWS_PER_TILE
        pltpu.sync_copy(z_hbm, acc_sh.at[pl.ds(r0, _ROWS_PER_TILE)])
        pltpu.sync_copy(src_hbm.at[pl.ds(w * _CH_PER_W, _CH_PER_W)], src_v)
        pltpu.sync_copy(dst_hbm.at[pl.ds(w * _CH_PER_W, _CH_PER_W)], dst_v)
        plsc.subcore_barrier()

        def gather(chunk, buf):
            pltpu.async_copy(h_hbm.at[src_v.at[chunk]], buf, gsem)

        def gwait(chunk, buf):
            pltpu.make_async_copy(h_hbm.at[src_v.at[chunk]], buf, gsem).wait()

        def scat(chunk, buf):
            pltpu.sync_copy(buf, acc_sh.at[dst_v.at[chunk]], add=True)

        gather(0, rows0)

        def body(j, carry):
            c0 = 2 * j
            gwait(c0, rows0)
            gather(c0 + 1, rows1)
            scat(c0, rows0)
            gwait(c0 + 1, rows1)
            gather(c0 + 2, rows0)
            scat(c0 + 1, rows1)
            return carry

        lax.fori_loop(0, (_CH_PER_W - 2) // 2, body, 0)
        last = _CH_PER_W - 2
        gwait(last, rows0)
        gather(last + 1, rows1)
        scat(last, rows0)
        gwait(last + 1, rows1)
        scat(last + 1, rows1)

        plsc.subcore_barrier()
        pltpu.sync_copy(acc_sh.at[pl.ds(r0, _ROWS_PER_TILE)],
                        out_hbm.at[c, pl.ds(r0, _ROWS_PER_TILE)])

    return agg_kernel(h, src2d, dst2d, zrows)


_PREC = lax.Precision.HIGHEST


def _bn_relu(y, g, b):
    m = jnp.mean(y, axis=0, keepdims=True)
    yc = y - m
    v = jnp.mean(yc * yc, axis=0, keepdims=True)
    return jnp.maximum(yc * lax.rsqrt(v + EPS_BN) * g + b, 0.0)


def _pool_project(h, bt_row, wlt, bl):
    oneh = (lax.broadcasted_iota(jnp.int32, (N_GRAPHS, N_NODES), 0)
            == bt_row).astype(jnp.float32)
    sums = jnp.dot(oneh, h, precision=_PREC)
    cnt = jnp.sum(oneh, axis=1, keepdims=True)
    pooled = sums / jnp.maximum(cnt, 1.0)
    return jnp.dot(pooled, wlt, precision=_PREC) + bl


def _first_body(x_ref, bt_ref, w1t_ref, b1_ref, g1_ref, be1_ref,
                w2t_ref, b2_ref, g2_ref, be2_ref, wlt_ref, bl_ref,
                h_ref, outp_ref):
    y = jnp.dot(x_ref[...], w1t_ref[...], precision=_PREC) + b1_ref[...]
    y = _bn_relu(y, g1_ref[...], be1_ref[...])
    y = jnp.dot(y, w2t_ref[...], precision=_PREC) + b2_ref[...]
    h = _bn_relu(y, g2_ref[...], be2_ref[...])
    h_ref[...] = h
    outp_ref[...] = _pool_project(h, bt_ref[...], wlt_ref[...], bl_ref[...])


def _layer_body(h_in_ref, agg_ref, eps_ref, bt_ref, w1t_ref, b1_ref, g1_ref,
                be1_ref, w2t_ref, b2_ref, g2_ref, be2_ref, wlt_ref, bl_ref,
                h_ref, outp_ref):
    u = (h_in_ref[...] * (1.0 + eps_ref[...])
         + agg_ref[0, :N_NODES, :] + agg_ref[1, :N_NODES, :])
    y = jnp.dot(u, w1t_ref[...], precision=_PREC) + b1_ref[...]
    y = _bn_relu(y, g1_ref[...], be1_ref[...])
    y = jnp.dot(y, w2t_ref[...], precision=_PREC) + b2_ref[...]
    h = _bn_relu(y, g2_ref[...], be2_ref[...])
    h_ref[...] = h
    outp_ref[...] = _pool_project(h, bt_ref[...], wlt_ref[...], bl_ref[...])


_DENSE_OUT = (
    jax.ShapeDtypeStruct((N_NODES, HID_DIM), jnp.float32),
    jax.ShapeDtypeStruct((N_GRAPHS, OUT_DIM), jnp.float32),
)

_first_call = pl.pallas_call(_first_body, out_shape=_DENSE_OUT)
_layer_call = pl.pallas_call(_layer_body, out_shape=_DENSE_OUT)


def _mlp_args(p):
    return (p["W1"].T, p["b1"].reshape(1, -1), p["g1"].reshape(1, -1),
            p["be1"].reshape(1, -1), p["W2"].T, p["b2"].reshape(1, -1),
            p["g2"].reshape(1, -1), p["be2"].reshape(1, -1))


def kernel(x, edge_index, batch, params):
    src = edge_index[0].reshape(_NCHUNKS, _CHUNK)
    dst = edge_index[1].reshape(_NCHUNKS, _CHUNK)
    bt_row = batch.reshape(1, N_NODES)
    zrows = jnp.zeros((_ROWS_PER_TILE, HID_DIM), jnp.float32)

    lin = params["lin"]
    h, out = _first_call(
        x, bt_row, *_mlp_args(params["first_h"]),
        lin[0]["W"].T, lin[0]["b"].reshape(1, -1))

    for layer in range(1, LAYERS):
        agg = _sc_edge_agg(h, src, dst, zrows)
        eps = params["eps"][layer - 1].reshape(1, 1)
        h, outp = _layer_call(
            h, agg, eps, bt_row, *_mlp_args(params["nns"][layer - 1]),
            lin[layer]["W"].T, lin[layer]["b"].reshape(1, -1))
        out = out + outp
    return out


# speedup vs baseline: 10.2748x; 1.1628x over previous
import functools

import jax
import jax.numpy as jnp
from jax import lax
from jax.experimental import pallas as pl
from jax.experimental.pallas import tpu as pltpu
from jax.experimental.pallas import tpu_sc as plsc

N_NODES = 10000
N_EDGES = 320000
N_GRAPHS = 64
IN_DIM = 128
HID_DIM = 64
OUT_DIM = 64
LAYERS = 4
EPS_BN = 1e-5


_NC = 2
_NS = 16
_NW = _NC * _NS
_CHUNK = 125
_NCHUNKS = N_EDGES // _CHUNK
_CH_PER_W = _NCHUNKS // _NW
_ROWS_PER_TILE = 632
_N_PAD = _ROWS_PER_TILE * _NS


def _sc_edge_agg(h, src2d, dst2d, zrows):
    mesh = plsc.VectorSubcoreMesh(core_axis_name="c", subcore_axis_name="s")

    @functools.partial(
        pl.kernel,
        mesh=mesh,
        out_type=jax.ShapeDtypeStruct((_NC, _N_PAD, HID_DIM), jnp.float32),
        scratch_types=[
            pltpu.VMEM((_CH_PER_W, _CHUNK), jnp.int32),
            pltpu.VMEM((_CH_PER_W, _CHUNK), jnp.int32),
            pltpu.VMEM((_CHUNK, HID_DIM), jnp.float32),
            pltpu.VMEM((_CHUNK, HID_DIM), jnp.float32),
            pltpu.VMEM_SHARED((_N_PAD, HID_DIM), jnp.float32),
            pltpu.SemaphoreType.DMA,
        ],
        compiler_params=pltpu.CompilerParams(use_tc_tiling_on_sc=False),
    )
    def agg_kernel(h_hbm, src_hbm, dst_hbm, z_hbm, out_hbm,
                   src_v, dst_v, rows0, rows1, acc_sh, gsem):
        c = lax.axis_index("c")
        s = lax.axis_index("s")
        w = s * _NC + c

        r0 = s * _ROWS_PER_TILE
        pltpu.sync_copy(z_hbm, acc_sh.at[pl.ds(r0, _ROWS_PER_TILE)])
        pltpu.sync_copy(src_hbm.at[pl.ds(w * _CH_PER_W, _CH_PER_W)], src_v)
        pltpu.sync_copy(dst_hbm.at[pl.ds(w * _CH_PER_W, _CH_PER_W)], dst_v)
        plsc.subcore_barrier()

        def gather(chunk, buf):
            pltpu.async_copy(h_hbm.at[src_v.at[chunk]], buf, gsem)

        def gwait(chunk, buf):
            pltpu.make_async_copy(h_hbm.at[src_v.at[chunk]], buf, gsem).wait()

        def scat(chunk, buf):
            pltpu.sync_copy(buf, acc_sh.at[dst_v.at[chunk]], add=True)

        gather(0, rows0)

        def body(j, carry):
            c0 = 2 * j
            gwait(c0, rows0)
            gather(c0 + 1, rows1)
            scat(c0, rows0)
            gwait(c0 + 1, rows1)
            gather(c0 + 2, rows0)
            scat(c0 + 1, rows1)
            return carry

        lax.fori_loop(0, (_CH_PER_W - 2) // 2, body, 0)
        last = _CH_PER_W - 2
        gwait(last, rows0)
        gather(last + 1, rows1)
        scat(last, rows0)
        gwait(last + 1, rows1)
        scat(last + 1, rows1)

        plsc.subcore_barrier()
        pltpu.sync_copy(acc_sh.at[pl.ds(r0, _ROWS_PER_TILE)],
                        out_hbm.at[c, pl.ds(r0, _ROWS_PER_TILE)])

    return agg_kernel(h, src2d, dst2d, zrows)


_PREC = lax.Precision.DEFAULT


def _bn_relu(y, g, b):
    m = jnp.mean(y, axis=0, keepdims=True)
    yc = y - m
    v = jnp.mean(yc * yc, axis=0, keepdims=True)
    return jnp.maximum(yc * lax.rsqrt(v + EPS_BN) * g + b, 0.0)


def _pool_project(h, bt_row, wlt, bl):
    oneh = (lax.broadcasted_iota(jnp.int32, (N_GRAPHS, N_NODES), 0)
            == bt_row).astype(jnp.float32)
    sums = jnp.dot(oneh, h, precision=_PREC)
    cnt = jnp.sum(oneh, axis=1, keepdims=True)
    pooled = sums / jnp.maximum(cnt, 1.0)
    return jnp.dot(pooled, wlt, precision=_PREC) + bl


def _first_body(x_ref, bt_ref, w1t_ref, b1_ref, g1_ref, be1_ref,
                w2t_ref, b2_ref, g2_ref, be2_ref, wlt_ref, bl_ref,
                h_ref, outp_ref):
    y = jnp.dot(x_ref[...], w1t_ref[...], precision=_PREC) + b1_ref[...]
    y = _bn_relu(y, g1_ref[...], be1_ref[...])
    y = jnp.dot(y, w2t_ref[...], precision=_PREC) + b2_ref[...]
    h = _bn_relu(y, g2_ref[...], be2_ref[...])
    h_ref[...] = h
    outp_ref[...] = _pool_project(h, bt_ref[...], wlt_ref[...], bl_ref[...])


def _layer_body(h_in_ref, agg_ref, eps_ref, bt_ref, w1t_ref, b1_ref, g1_ref,
                be1_ref, w2t_ref, b2_ref, g2_ref, be2_ref, wlt_ref, bl_ref,
                h_ref, outp_ref):
    u = (h_in_ref[...] * (1.0 + eps_ref[...])
         + agg_ref[0, :N_NODES, :] + agg_ref[1, :N_NODES, :])
    y = jnp.dot(u, w1t_ref[...], precision=_PREC) + b1_ref[...]
    y = _bn_relu(y, g1_ref[...], be1_ref[...])
    y = jnp.dot(y, w2t_ref[...], precision=_PREC) + b2_ref[...]
    h = _bn_relu(y, g2_ref[...], be2_ref[...])
    h_ref[...] = h
    outp_ref[...] = _pool_project(h, bt_ref[...], wlt_ref[...], bl_ref[...])


_DENSE_OUT = (
    jax.ShapeDtypeStruct((N_NODES, HID_DIM), jnp.float32),
    jax.ShapeDtypeStruct((N_GRAPHS, OUT_DIM), jnp.float32),
)

_first_call = pl.pallas_call(_first_body, out_shape=_DENSE_OUT)
_layer_call = pl.pallas_call(_layer_body, out_shape=_DENSE_OUT)


def _mlp_args(p):
    return (p["W1"].T, p["b1"].reshape(1, -1), p["g1"].reshape(1, -1),
            p["be1"].reshape(1, -1), p["W2"].T, p["b2"].reshape(1, -1),
            p["g2"].reshape(1, -1), p["be2"].reshape(1, -1))


def kernel(x, edge_index, batch, params):
    src = edge_index[0].reshape(_NCHUNKS, _CHUNK)
    dst = edge_index[1].reshape(_NCHUNKS, _CHUNK)
    bt_row = batch.reshape(1, N_NODES)
    zrows = jnp.zeros((_ROWS_PER_TILE, HID_DIM), jnp.float32)

    lin = params["lin"]
    h, out = _first_call(
        x, bt_row, *_mlp_args(params["first_h"]),
        lin[0]["W"].T, lin[0]["b"].reshape(1, -1))

    for layer in range(1, LAYERS):
        agg = _sc_edge_agg(h, src, dst, zrows)
        eps = params["eps"][layer - 1].reshape(1, 1)
        h, outp = _layer_call(
            h, agg, eps, bt_row, *_mlp_args(params["nns"][layer - 1]),
            lin[layer]["W"].T, lin[layer]["b"].reshape(1, -1))
        out = out + outp
    return out


# speedup vs baseline: 12.5176x; 1.2183x over previous
import functools

import jax
import jax.numpy as jnp
from jax import lax
from jax.experimental import pallas as pl
from jax.experimental.pallas import tpu as pltpu
from jax.experimental.pallas import tpu_sc as plsc

N_NODES = 10000
N_EDGES = 320000
N_GRAPHS = 64
IN_DIM = 128
HID_DIM = 64
OUT_DIM = 64
LAYERS = 4
EPS_BN = 1e-5


_NC = 2
_NS = 16
_NW = _NC * _NS
_CHUNK = 125
_NCHUNKS = N_EDGES // _CHUNK
_CH_PER_W = _NCHUNKS // _NW
_ROWS_PER_TILE = 632
_N_PAD = _ROWS_PER_TILE * _NS


def _sc_edge_agg(h, src2d, dst2d, zrows):
    mesh = plsc.VectorSubcoreMesh(core_axis_name="c", subcore_axis_name="s")

    @functools.partial(
        pl.kernel,
        mesh=mesh,
        out_type=jax.ShapeDtypeStruct((_NC, _N_PAD, HID_DIM), jnp.float32),
        scratch_types=[
            pltpu.VMEM((_CH_PER_W, _CHUNK), jnp.int32),
            pltpu.VMEM((_CH_PER_W, _CHUNK), jnp.int32),
            [pltpu.VMEM((_CHUNK, HID_DIM), jnp.float32) for _ in range(4)],
            [pltpu.SemaphoreType.DMA for _ in range(4)],
            [pltpu.SemaphoreType.DMA for _ in range(4)],
            pltpu.VMEM_SHARED((_N_PAD, HID_DIM), jnp.float32),
        ],
        compiler_params=pltpu.CompilerParams(use_tc_tiling_on_sc=False),
    )
    def agg_kernel(h_hbm, src_hbm, dst_hbm, z_hbm, out_hbm,
                   src_v, dst_v, rows, gsems, ssems, acc_sh):
        c = lax.axis_index("c")
        s = lax.axis_index("s")
        w = s * _NC + c

        r0 = s * _ROWS_PER_TILE
        pltpu.sync_copy(z_hbm, acc_sh.at[pl.ds(r0, _ROWS_PER_TILE)])
        pltpu.sync_copy(src_hbm.at[pl.ds(w * _CH_PER_W, _CH_PER_W)], src_v)
        pltpu.sync_copy(dst_hbm.at[pl.ds(w * _CH_PER_W, _CH_PER_W)], dst_v)
        plsc.subcore_barrier()

        def gstart(chunk, p):
            pltpu.async_copy(h_hbm.at[src_v.at[chunk]], rows[p], gsems[p])

        def gwait(chunk, p):
            pltpu.make_async_copy(h_hbm.at[src_v.at[chunk]], rows[p],
                                  gsems[p]).wait()

        def sstart(chunk, p):
            pltpu.async_copy(rows[p], acc_sh.at[dst_v.at[chunk]], ssems[p],
                             add=True)

        def swait(chunk, p):
            pltpu.make_async_copy(rows[p], acc_sh.at[dst_v.at[chunk]],
                                  ssems[p]).wait()

        gstart(0, 0)
        for cc in range(3):
            gstart(cc + 1, (cc + 1) % 4)
            gwait(cc, cc % 4)
            sstart(cc, cc % 4)

        def body(j, carry):
            c0 = 3 + 4 * j
            for k in range(4):
                ck = c0 + k
                p = (3 + k) % 4
                q = k
                swait(ck - 3, q)
                gstart(ck + 1, q)
                gwait(ck, p)
                sstart(ck, p)
            return carry

        lax.fori_loop(0, (_CH_PER_W - 4) // 4, body, 0)
        last = _CH_PER_W - 1
        gwait(last, last % 4)
        sstart(last, last % 4)
        for cc in range(last - 3, last + 1):
            swait(cc, cc % 4)

        plsc.subcore_barrier()
        pltpu.sync_copy(acc_sh.at[pl.ds(r0, _ROWS_PER_TILE)],
                        out_hbm.at[c, pl.ds(r0, _ROWS_PER_TILE)])

    return agg_kernel(h, src2d, dst2d, zrows)


_PREC = lax.Precision.DEFAULT


def _bn_relu(y, g, b):
    m = jnp.mean(y, axis=0, keepdims=True)
    yc = y - m
    v = jnp.mean(yc * yc, axis=0, keepdims=True)
    return jnp.maximum(yc * lax.rsqrt(v + EPS_BN) * g + b, 0.0)


def _first_body(x_ref, w1t_ref, b1_ref, g1_ref, be1_ref,
                w2t_ref, b2_ref, g2_ref, be2_ref, h_ref):
    y = jnp.dot(x_ref[...], w1t_ref[...], precision=_PREC) + b1_ref[...]
    y = _bn_relu(y, g1_ref[...], be1_ref[...])
    y = jnp.dot(y, w2t_ref[...], precision=_PREC) + b2_ref[...]
    h_ref[...] = _bn_relu(y, g2_ref[...], be2_ref[...])


def _layer_body(h_in_ref, agg_ref, eps_ref, w1t_ref, b1_ref, g1_ref,
                be1_ref, w2t_ref, b2_ref, g2_ref, be2_ref, h_ref):
    u = (h_in_ref[...] * (1.0 + eps_ref[...])
         + agg_ref[0, :N_NODES, :] + agg_ref[1, :N_NODES, :])
    y = jnp.dot(u, w1t_ref[...], precision=_PREC) + b1_ref[...]
    y = _bn_relu(y, g1_ref[...], be1_ref[...])
    y = jnp.dot(y, w2t_ref[...], precision=_PREC) + b2_ref[...]
    h_ref[...] = _bn_relu(y, g2_ref[...], be2_ref[...])


def _pool_body(bt_ref, h1_ref, h2_ref, h3_ref, h4_ref, wlts_ref, bls_ref,
               out_ref):
    oneh = (lax.broadcasted_iota(jnp.int32, (N_GRAPHS, N_NODES), 0)
            == bt_ref[...]).astype(jnp.float32)
    inv_cnt = 1.0 / jnp.maximum(jnp.sum(oneh, axis=1, keepdims=True), 1.0)
    acc = bls_ref[0] + bls_ref[1] + bls_ref[2] + bls_ref[3]
    for i, h_ref in enumerate((h1_ref, h2_ref, h3_ref, h4_ref)):
        pooled = jnp.dot(oneh, h_ref[...], precision=_PREC) * inv_cnt
        acc = acc + jnp.dot(pooled, wlts_ref[i], precision=_PREC)
    out_ref[...] = acc


_H_OUT = jax.ShapeDtypeStruct((N_NODES, HID_DIM), jnp.float32)

_first_call = pl.pallas_call(_first_body, out_shape=_H_OUT)
_layer_call = pl.pallas_call(_layer_body, out_shape=_H_OUT)
_pool_call = pl.pallas_call(
    _pool_body,
    out_shape=jax.ShapeDtypeStruct((N_GRAPHS, OUT_DIM), jnp.float32))


def _mlp_args(p):
    return (p["W1"].T, p["b1"].reshape(1, -1), p["g1"].reshape(1, -1),
            p["be1"].reshape(1, -1), p["W2"].T, p["b2"].reshape(1, -1),
            p["g2"].reshape(1, -1), p["be2"].reshape(1, -1))


def kernel(x, edge_index, batch, params):
    src = edge_index[0].reshape(_NCHUNKS, _CHUNK)
    dst = edge_index[1].reshape(_NCHUNKS, _CHUNK)
    bt_row = batch.reshape(1, N_NODES)
    zrows = jnp.zeros((_ROWS_PER_TILE, HID_DIM), jnp.float32)

    lin = params["lin"]
    hs = [_first_call(x, *_mlp_args(params["first_h"]))]
    for layer in range(1, LAYERS):
        agg = _sc_edge_agg(hs[-1], src, dst, zrows)
        eps = params["eps"][layer - 1].reshape(1, 1)
        hs.append(_layer_call(hs[-1], agg, eps,
                              *_mlp_args(params["nns"][layer - 1])))

    wlts = jnp.stack([lin[i]["W"].T for i in range(LAYERS)])
    bls = jnp.stack([lin[i]["b"].reshape(1, -1) for i in range(LAYERS)])
    return _pool_call(bt_row, *hs, wlts, bls)


# speedup vs baseline: 13.5681x; 1.0839x over previous
import functools

import jax
import jax.numpy as jnp
from jax import lax
from jax.experimental import pallas as pl
from jax.experimental.pallas import tpu as pltpu
from jax.experimental.pallas import tpu_sc as plsc

N_NODES = 10000
N_EDGES = 320000
N_GRAPHS = 64
IN_DIM = 128
HID_DIM = 64
OUT_DIM = 64
LAYERS = 4
EPS_BN = 1e-5


_NC = 2
_NS = 16
_NW = _NC * _NS
_CHUNK = 125
_NCHUNKS = N_EDGES // _CHUNK
_CH_PER_W = _NCHUNKS // _NW
_ROWS_PER_TILE = 632
_N_PAD = _ROWS_PER_TILE * _NS


_DEPTH = 8


def _sc_edge_agg(h, edges3d, zrows):
    mesh = plsc.VectorSubcoreMesh(core_axis_name="c", subcore_axis_name="s")

    @functools.partial(
        pl.kernel,
        mesh=mesh,
        out_type=jax.ShapeDtypeStruct((_NC, _N_PAD, HID_DIM), jnp.float32),
        scratch_types=[
            pltpu.VMEM((_CH_PER_W, _CHUNK), jnp.int32),
            pltpu.VMEM((_CH_PER_W, _CHUNK), jnp.int32),
            [pltpu.VMEM((_CHUNK, HID_DIM), jnp.float32) for _ in range(_DEPTH)],
            [pltpu.SemaphoreType.DMA for _ in range(_DEPTH)],
            [pltpu.SemaphoreType.DMA for _ in range(_DEPTH)],
            pltpu.VMEM_SHARED((_N_PAD, HID_DIM), jnp.float32),
        ],
        compiler_params=pltpu.CompilerParams(use_tc_tiling_on_sc=False),
    )
    def agg_kernel(h_hbm, e_hbm, z_hbm, out_hbm,
                   src_v, dst_v, rows, gsems, ssems, acc_sh):
        c = lax.axis_index("c")
        s = lax.axis_index("s")
        w = s * _NC + c

        r0 = s * _ROWS_PER_TILE
        pltpu.sync_copy(z_hbm, acc_sh.at[pl.ds(r0, _ROWS_PER_TILE)])
        pltpu.sync_copy(e_hbm.at[0, pl.ds(w * _CH_PER_W, _CH_PER_W)], src_v)
        pltpu.sync_copy(e_hbm.at[1, pl.ds(w * _CH_PER_W, _CH_PER_W)], dst_v)
        plsc.subcore_barrier()

        def gstart(chunk, p):
            pltpu.async_copy(h_hbm.at[src_v.at[chunk]], rows[p], gsems[p])

        def gwait(chunk, p):
            pltpu.make_async_copy(h_hbm.at[src_v.at[chunk]], rows[p],
                                  gsems[p]).wait()

        def sstart(chunk, p):
            pltpu.async_copy(rows[p], acc_sh.at[dst_v.at[chunk]], ssems[p],
                             add=True)

        def swait(chunk, p):
            pltpu.make_async_copy(rows[p], acc_sh.at[dst_v.at[chunk]],
                                  ssems[p]).wait()

        gstart(0, 0)
        gstart(1, 1)
        for cc in range(_DEPTH - 2):
            gstart(cc + 2, cc + 2)
            gwait(cc, cc)
            sstart(cc, cc)

        def body(j, carry):
            c0 = (_DEPTH - 2) + _DEPTH * j
            for k in range(_DEPTH):
                ck = c0 + k
                p = (_DEPTH - 2 + k) % _DEPTH
                q = k
                swait(ck - (_DEPTH - 2), q)
                gstart(ck + 2, q)
                gwait(ck, p)
                sstart(ck, p)
            return carry

        n_steady = _CH_PER_W - _DEPTH
        lax.fori_loop(0, n_steady // _DEPTH, body, 0)
        for ck in range(_CH_PER_W - 2, _CH_PER_W):
            gwait(ck, ck % _DEPTH)
            sstart(ck, ck % _DEPTH)
        for ck in range(_CH_PER_W - _DEPTH, _CH_PER_W):
            swait(ck, ck % _DEPTH)

        plsc.subcore_barrier()
        pltpu.sync_copy(acc_sh.at[pl.ds(r0, _ROWS_PER_TILE)],
                        out_hbm.at[c, pl.ds(r0, _ROWS_PER_TILE)])

    return agg_kernel(h, edges3d, zrows)


_PREC = lax.Precision.DEFAULT


def _bn_relu(y, g, b):
    m = jnp.mean(y, axis=0, keepdims=True)
    yc = y - m
    v = jnp.mean(yc * yc, axis=0, keepdims=True)
    return jnp.maximum(yc * lax.rsqrt(v + EPS_BN) * g + b, 0.0)


def _first_body(x_ref, w1t_ref, b1_ref, g1_ref, be1_ref,
                w2t_ref, b2_ref, g2_ref, be2_ref, h_ref):
    y = jnp.dot(x_ref[...], w1t_ref[...], precision=_PREC) + b1_ref[...]
    y = _bn_relu(y, g1_ref[...], be1_ref[...])
    y = jnp.dot(y, w2t_ref[...], precision=_PREC) + b2_ref[...]
    h_ref[...] = _bn_relu(y, g2_ref[...], be2_ref[...])


def _layer_body(h_in_ref, agg_ref, eps_ref, w1t_ref, b1_ref, g1_ref,
                be1_ref, w2t_ref, b2_ref, g2_ref, be2_ref, h_ref):
    u = (h_in_ref[...] * (1.0 + eps_ref[...])
         + agg_ref[0, :N_NODES, :] + agg_ref[1, :N_NODES, :])
    y = jnp.dot(u, w1t_ref[...], precision=_PREC) + b1_ref[...]
    y = _bn_relu(y, g1_ref[...], be1_ref[...])
    y = jnp.dot(y, w2t_ref[...], precision=_PREC) + b2_ref[...]
    h_ref[...] = _bn_relu(y, g2_ref[...], be2_ref[...])


def _pool_body(bt_ref, h1_ref, h2_ref, h3_ref, h4_ref, wlts_ref, bls_ref,
               out_ref):
    oneh = (lax.broadcasted_iota(jnp.int32, (N_GRAPHS, N_NODES), 0)
            == bt_ref[...][None, :]).astype(jnp.float32)
    inv_cnt = 1.0 / jnp.maximum(jnp.sum(oneh, axis=1, keepdims=True), 1.0)
    acc = bls_ref[0] + bls_ref[1] + bls_ref[2] + bls_ref[3]
    for i, h_ref in enumerate((h1_ref, h2_ref, h3_ref, h4_ref)):
        pooled = jnp.dot(oneh, h_ref[...], precision=_PREC) * inv_cnt
        acc = acc + jnp.dot(pooled, wlts_ref[i], precision=_PREC)
    out_ref[...] = acc


_H_OUT = jax.ShapeDtypeStruct((N_NODES, HID_DIM), jnp.float32)

_first_call = pl.pallas_call(_first_body, out_shape=_H_OUT)
_layer_call = pl.pallas_call(_layer_body, out_shape=_H_OUT)
_pool_call = pl.pallas_call(
    _pool_body,
    out_shape=jax.ShapeDtypeStruct((N_GRAPHS, OUT_DIM), jnp.float32))


def _mlp_args(p):
    return (p["W1"].T, p["b1"].reshape(1, -1), p["g1"].reshape(1, -1),
            p["be1"].reshape(1, -1), p["W2"].T, p["b2"].reshape(1, -1),
            p["g2"].reshape(1, -1), p["be2"].reshape(1, -1))


def kernel(x, edge_index, batch, params):
    edges3d = edge_index.reshape(2, _NCHUNKS, _CHUNK)
    zrows = jnp.zeros((_ROWS_PER_TILE, HID_DIM), jnp.float32)

    lin = params["lin"]
    hs = [_first_call(x, *_mlp_args(params["first_h"]))]
    for layer in range(1, LAYERS):
        agg = _sc_edge_agg(hs[-1], edges3d, zrows)
        eps = params["eps"][layer - 1].reshape(1, 1)
        hs.append(_layer_call(hs[-1], agg, eps,
                              *_mlp_args(params["nns"][layer - 1])))

    wlts = jnp.stack([lin[i]["W"].T for i in range(LAYERS)])
    bls = jnp.stack([lin[i]["b"].reshape(1, -1) for i in range(LAYERS)])
    return _pool_call(batch, *hs, wlts, bls)


# speedup vs baseline: 16.1650x; 1.1914x over previous
import functools

import jax
import jax.numpy as jnp
from jax import lax
from jax.experimental import pallas as pl
from jax.experimental.pallas import tpu as pltpu
from jax.experimental.pallas import tpu_sc as plsc

N_NODES = 10000
N_EDGES = 320000
N_GRAPHS = 64
IN_DIM = 128
HID_DIM = 64
OUT_DIM = 64
LAYERS = 4
EPS_BN = 1e-5


_NC = 2
_NS = 16
_NW = _NC * _NS
_CHUNK = 125
_NCHUNKS = N_EDGES // _CHUNK
_CH_PER_W = _NCHUNKS // _NW
_ROWS_PER_TILE = 632
_N_PAD = _ROWS_PER_TILE * _NS


_DEPTH = 8


def _sc_edge_agg(h, edges3d, zrows):
    mesh = plsc.VectorSubcoreMesh(core_axis_name="c", subcore_axis_name="s")

    @functools.partial(
        pl.kernel,
        mesh=mesh,
        out_type=jax.ShapeDtypeStruct((_NC, _N_PAD, HID_DIM), jnp.float32),
        scratch_types=[
            pltpu.VMEM((_CH_PER_W, _CHUNK), jnp.int32),
            pltpu.VMEM((_CH_PER_W, _CHUNK), jnp.int32),
            [pltpu.VMEM((_CHUNK, HID_DIM), jnp.float32) for _ in range(_DEPTH)],
            [pltpu.SemaphoreType.DMA for _ in range(_DEPTH)],
            [pltpu.SemaphoreType.DMA for _ in range(_DEPTH)],
            pltpu.VMEM_SHARED((_N_PAD, HID_DIM), jnp.float32),
        ],
        compiler_params=pltpu.CompilerParams(use_tc_tiling_on_sc=False),
    )
    def agg_kernel(h_hbm, e_hbm, z_hbm, out_hbm,
                   src_v, dst_v, rows, gsems, ssems, acc_sh):
        c = lax.axis_index("c")
        s = lax.axis_index("s")
        w = s * _NC + c

        r0 = s * _ROWS_PER_TILE
        pltpu.sync_copy(z_hbm, acc_sh.at[pl.ds(r0, _ROWS_PER_TILE)])
        pltpu.sync_copy(e_hbm.at[0, pl.ds(w * _CH_PER_W, _CH_PER_W)], src_v)
        pltpu.sync_copy(e_hbm.at[1, pl.ds(w * _CH_PER_W, _CH_PER_W)], dst_v)
        plsc.subcore_barrier()

        def gstart(chunk, p):
            pltpu.async_copy(h_hbm.at[src_v.at[chunk]], rows[p], gsems[p])

        def gwait(chunk, p):
            pltpu.make_async_copy(h_hbm.at[src_v.at[chunk]], rows[p],
                                  gsems[p]).wait()

        def sstart(chunk, p):
            pltpu.async_copy(rows[p], acc_sh.at[dst_v.at[chunk]], ssems[p],
                             add=True)

        def swait(chunk, p):
            pltpu.make_async_copy(rows[p], acc_sh.at[dst_v.at[chunk]],
                                  ssems[p]).wait()

        gstart(0, 0)
        gstart(1, 1)
        for cc in range(_DEPTH - 2):
            gstart(cc + 2, cc + 2)
            gwait(cc, cc)
            sstart(cc, cc)

        def body(j, carry):
            c0 = (_DEPTH - 2) + _DEPTH * j
            for k in range(_DEPTH):
                ck = c0 + k
                p = (_DEPTH - 2 + k) % _DEPTH
                q = k
                swait(ck - (_DEPTH - 2), q)
                gstart(ck + 2, q)
                gwait(ck, p)
                sstart(ck, p)
            return carry

        n_steady = _CH_PER_W - _DEPTH
        lax.fori_loop(0, n_steady // _DEPTH, body, 0)
        for ck in range(_CH_PER_W - 2, _CH_PER_W):
            gwait(ck, ck % _DEPTH)
            sstart(ck, ck % _DEPTH)
        for ck in range(_CH_PER_W - _DEPTH, _CH_PER_W):
            swait(ck, ck % _DEPTH)

        plsc.subcore_barrier()
        pltpu.sync_copy(acc_sh.at[pl.ds(r0, _ROWS_PER_TILE)],
                        out_hbm.at[c, pl.ds(r0, _ROWS_PER_TILE)])

    return agg_kernel(h, edges3d, zrows)


_PREC = lax.Precision.DEFAULT


_NP = N_NODES // 2
_PACK = 2 * HID_DIM


_HI = lax.Precision.HIGHEST


def _bn_relu_packed(y, fold_ref, unfold_ref, g2, be2):
    m = jnp.dot(jnp.mean(y, axis=0, keepdims=True), fold_ref[...],
                precision=_HI) * 0.5
    yc = y - jnp.dot(m, unfold_ref[...], precision=_HI)
    v = jnp.dot(jnp.mean(yc * yc, axis=0, keepdims=True), fold_ref[...],
                precision=_HI) * 0.5
    vb = jnp.dot(v, unfold_ref[...], precision=_HI)
    return jnp.maximum(yc * lax.rsqrt(vb + EPS_BN) * g2 + be2, 0.0)


def _first_body(x_ref, fold_ref, unfold_ref, w1t_ref, b1_ref, g1_ref,
                be1_ref, w2t_ref, b2_ref, g2_ref, be2_ref, h_ref):
    y = jnp.dot(x_ref[...], w1t_ref[...], precision=_PREC) + b1_ref[...]
    y = _bn_relu_packed(y, fold_ref, unfold_ref, g1_ref[...], be1_ref[...])
    y = jnp.dot(y, w2t_ref[...], precision=_PREC) + b2_ref[...]
    h_ref[...] = _bn_relu_packed(y, fold_ref, unfold_ref, g2_ref[...],
                                 be2_ref[...])


def _layer_body(h_in_ref, agg_ref, eps_ref, fold_ref, unfold_ref, w1t_ref,
                b1_ref, g1_ref, be1_ref, w2t_ref, b2_ref, g2_ref, be2_ref,
                h_ref):
    u = (h_in_ref[...] * (1.0 + eps_ref[...])
         + agg_ref[0, :_NP, :] + agg_ref[1, :_NP, :])
    y = jnp.dot(u, w1t_ref[...], precision=_PREC) + b1_ref[...]
    y = _bn_relu_packed(y, fold_ref, unfold_ref, g1_ref[...], be1_ref[...])
    y = jnp.dot(y, w2t_ref[...], precision=_PREC) + b2_ref[...]
    h_ref[...] = _bn_relu_packed(y, fold_ref, unfold_ref, g2_ref[...],
                                 be2_ref[...])


def _pool_body(be_ref, bo_ref, se_ref, so_ref, h1_ref, h2_ref, h3_ref,
               h4_ref, wlts_ref, bls_ref, out_ref):
    ae = (lax.broadcasted_iota(jnp.int32, (N_GRAPHS, _NP), 0)
          == be_ref[...]).astype(jnp.float32)
    ao = (lax.broadcasted_iota(jnp.int32, (N_GRAPHS, _NP), 0)
          == bo_ref[...]).astype(jnp.float32)
    cnt = (jnp.sum(ae, axis=1, keepdims=True)
           + jnp.sum(ao, axis=1, keepdims=True))
    inv_cnt = 1.0 / jnp.maximum(cnt, 1.0)
    acc = bls_ref[0] + bls_ref[1] + bls_ref[2] + bls_ref[3]
    for i, h_ref in enumerate((h1_ref, h2_ref, h3_ref, h4_ref)):
        hp = h_ref[...]
        sums = (jnp.dot(jnp.dot(ae, hp, precision=_PREC), se_ref[...],
                        precision=_PREC)
                + jnp.dot(jnp.dot(ao, hp, precision=_PREC), so_ref[...],
                          precision=_PREC))
        acc = acc + jnp.dot(sums * inv_cnt, wlts_ref[i], precision=_PREC)
    out_ref[...] = acc


_H_OUT = jax.ShapeDtypeStruct((_NP, _PACK), jnp.float32)

_first_call = pl.pallas_call(_first_body, out_shape=_H_OUT)
_layer_call = pl.pallas_call(_layer_body, out_shape=_H_OUT)
_pool_call = pl.pallas_call(
    _pool_body,
    out_shape=jax.ShapeDtypeStruct((N_GRAPHS, OUT_DIM), jnp.float32))


def _blockdiag(w):
    z = jnp.zeros_like(w)
    return jnp.block([[w, z], [z, w]])


def _tile2(v):
    return jnp.concatenate([v, v]).reshape(1, -1)


def _mlp_args(p):
    return (_blockdiag(p["W1"].T), _tile2(p["b1"]), _tile2(p["g1"]),
            _tile2(p["be1"]), _blockdiag(p["W2"].T), _tile2(p["b2"]),
            _tile2(p["g2"]), _tile2(p["be2"]))


def kernel(x, edge_index, batch, params):
    edges3d = edge_index.reshape(2, _NCHUNKS, _CHUNK)
    zrows = jnp.zeros((_ROWS_PER_TILE, HID_DIM), jnp.float32)

    eye = jnp.eye(HID_DIM, dtype=jnp.float32)
    zed = jnp.zeros((HID_DIM, HID_DIM), jnp.float32)
    fold = jnp.concatenate([eye, eye], axis=0)
    unfold = jnp.concatenate([eye, eye], axis=1)
    se = jnp.concatenate([eye, zed], axis=0)
    so = jnp.concatenate([zed, eye], axis=0)

    x_p = x.reshape(_NP, 2 * IN_DIM)
    b2d = batch.reshape(_NP, 2)
    b_even = b2d[:, 0].reshape(1, _NP)
    b_odd = b2d[:, 1].reshape(1, _NP)

    lin = params["lin"]
    hs = [_first_call(x_p, fold, unfold, *_mlp_args(params["first_h"]))]
    for layer in range(1, LAYERS):
        agg = _sc_edge_agg(hs[-1].reshape(N_NODES, HID_DIM), edges3d, zrows)
        agg_p = agg.reshape(_NC, _N_PAD // 2, _PACK)
        eps = params["eps"][layer - 1].reshape(1, 1)
        hs.append(_layer_call(hs[-1], agg_p, eps, fold, unfold,
                              *_mlp_args(params["nns"][layer - 1])))

    wlts = jnp.stack([lin[i]["W"].T for i in range(LAYERS)])
    bls = jnp.stack([lin[i]["b"].reshape(1, -1) for i in range(LAYERS)])
    return _pool_call(b_even, b_odd, se, so, *hs, wlts, bls)


# speedup vs baseline: 16.5760x; 1.0254x over previous
import functools

import jax
import jax.numpy as jnp
from jax import lax
from jax.experimental import pallas as pl
from jax.experimental.pallas import tpu as pltpu
from jax.experimental.pallas import tpu_sc as plsc

N_NODES = 10000
N_EDGES = 320000
N_GRAPHS = 64
IN_DIM = 128
HID_DIM = 64
OUT_DIM = 64
LAYERS = 4
EPS_BN = 1e-5


_NC = 2
_NS = 16
_NW = _NC * _NS
_CHUNK = 125
_NCHUNKS = N_EDGES // _CHUNK
_CH_PER_W = _NCHUNKS // _NW
_ROWS_PER_TILE = 632
_N_PAD = _ROWS_PER_TILE * _NS


_DEPTH = 8


def _sc_edge_agg(h, edges3d, zrows):
    mesh = plsc.VectorSubcoreMesh(core_axis_name="c", subcore_axis_name="s")

    @functools.partial(
        pl.kernel,
        mesh=mesh,
        out_type=jax.ShapeDtypeStruct((_NC, _N_PAD, HID_DIM), jnp.float32),
        scratch_types=[
            pltpu.VMEM((_CH_PER_W, _CHUNK), jnp.int32),
            pltpu.VMEM((_CH_PER_W, _CHUNK), jnp.int32),
            [pltpu.VMEM((_CHUNK, HID_DIM), jnp.float32) for _ in range(_DEPTH)],
            [pltpu.SemaphoreType.DMA for _ in range(_DEPTH)],
            [pltpu.SemaphoreType.DMA for _ in range(_DEPTH)],
            pltpu.VMEM_SHARED((_N_PAD, HID_DIM), jnp.float32),
        ],
        compiler_params=pltpu.CompilerParams(use_tc_tiling_on_sc=False),
    )
    def agg_kernel(h_hbm, e_hbm, z_hbm, out_hbm,
                   src_v, dst_v, rows, gsems, ssems, acc_sh):
        c = lax.axis_index("c")
        s = lax.axis_index("s")
        w = s * _NC + c

        r0 = s * _ROWS_PER_TILE
        zslab = acc_sh.at[pl.ds(r0, _ROWS_PER_TILE)]
        pltpu.async_copy(z_hbm, zslab, ssems[_DEPTH - 1])
        pltpu.sync_copy(e_hbm.at[0, pl.ds(w * _CH_PER_W, _CH_PER_W)], src_v)
        pltpu.sync_copy(e_hbm.at[1, pl.ds(w * _CH_PER_W, _CH_PER_W)], dst_v)

        def gstart(chunk, p):
            pltpu.async_copy(h_hbm.at[src_v.at[chunk]], rows[p], gsems[p])

        def gwait(chunk, p):
            pltpu.make_async_copy(h_hbm.at[src_v.at[chunk]], rows[p],
                                  gsems[p]).wait()

        def sstart(chunk, p):
            pltpu.async_copy(rows[p], acc_sh.at[dst_v.at[chunk]], ssems[p],
                             add=True)

        def swait(chunk, p):
            pltpu.make_async_copy(rows[p], acc_sh.at[dst_v.at[chunk]],
                                  ssems[p]).wait()

        for cc in range(_DEPTH):
            gstart(cc, cc)
        pltpu.make_async_copy(z_hbm, zslab, ssems[_DEPTH - 1]).wait()
        plsc.subcore_barrier()
        for cc in range(_DEPTH - 2):
            gwait(cc, cc)
            sstart(cc, cc)

        def body(j, carry):
            c0 = (_DEPTH - 2) + _DEPTH * j
            for k in range(_DEPTH):
                ck = c0 + k
                p = (_DEPTH - 2 + k) % _DEPTH
                q = k
                swait(ck - (_DEPTH - 2), q)
                gstart(ck + 2, q)
                gwait(ck, p)
                sstart(ck, p)
            return carry

        n_steady = _CH_PER_W - _DEPTH
        lax.fori_loop(0, n_steady // _DEPTH, body, 0)
        for ck in range(_CH_PER_W - 2, _CH_PER_W):
            gwait(ck, ck % _DEPTH)
            sstart(ck, ck % _DEPTH)
        for ck in range(_CH_PER_W - _DEPTH, _CH_PER_W):
            swait(ck, ck % _DEPTH)

        plsc.subcore_barrier()
        pltpu.sync_copy(acc_sh.at[pl.ds(r0, _ROWS_PER_TILE)],
                        out_hbm.at[c, pl.ds(r0, _ROWS_PER_TILE)])

    return agg_kernel(h, edges3d, zrows)


_PREC = lax.Precision.DEFAULT


_NP = N_NODES // 2
_PACK = 2 * HID_DIM


_HI = lax.Precision.HIGHEST


def _bn_relu_packed(y, fold_ref, unfold_ref, g2, be2):
    m = jnp.dot(jnp.mean(y, axis=0, keepdims=True), fold_ref[...],
                precision=_HI) * 0.5
    yc = y - jnp.dot(m, unfold_ref[...], precision=_HI)
    v = jnp.dot(jnp.mean(yc * yc, axis=0, keepdims=True), fold_ref[...],
                precision=_HI) * 0.5
    vb = jnp.dot(v, unfold_ref[...], precision=_HI)
    return jnp.maximum(yc * lax.rsqrt(vb + EPS_BN) * g2 + be2, 0.0)


def _first_body(x_ref, fold_ref, unfold_ref, w1t_ref, b1_ref, g1_ref,
                be1_ref, w2t_ref, b2_ref, g2_ref, be2_ref, h_ref):
    y = jnp.dot(x_ref[...], w1t_ref[...], precision=_PREC) + b1_ref[...]
    y = _bn_relu_packed(y, fold_ref, unfold_ref, g1_ref[...], be1_ref[...])
    y = jnp.dot(y, w2t_ref[...], precision=_PREC) + b2_ref[...]
    h_ref[...] = _bn_relu_packed(y, fold_ref, unfold_ref, g2_ref[...],
                                 be2_ref[...])


def _layer_body(h_in_ref, agg_ref, eps_ref, fold_ref, unfold_ref, w1t_ref,
                b1_ref, g1_ref, be1_ref, w2t_ref, b2_ref, g2_ref, be2_ref,
                h_ref):
    u = (h_in_ref[...] * (1.0 + eps_ref[...])
         + agg_ref[0, :_NP, :] + agg_ref[1, :_NP, :])
    y = jnp.dot(u, w1t_ref[...], precision=_PREC) + b1_ref[...]
    y = _bn_relu_packed(y, fold_ref, unfold_ref, g1_ref[...], be1_ref[...])
    y = jnp.dot(y, w2t_ref[...], precision=_PREC) + b2_ref[...]
    h_ref[...] = _bn_relu_packed(y, fold_ref, unfold_ref, g2_ref[...],
                                 be2_ref[...])


def _pool_body(be_ref, bo_ref, se_ref, so_ref, h1_ref, h2_ref, h3_ref,
               h4_ref, wlts_ref, bls_ref, out_ref):
    ae = (lax.broadcasted_iota(jnp.int32, (N_GRAPHS, _NP), 0)
          == be_ref[...]).astype(jnp.float32)
    ao = (lax.broadcasted_iota(jnp.int32, (N_GRAPHS, _NP), 0)
          == bo_ref[...]).astype(jnp.float32)
    cnt = (jnp.sum(ae, axis=1, keepdims=True)
           + jnp.sum(ao, axis=1, keepdims=True))
    inv_cnt = 1.0 / jnp.maximum(cnt, 1.0)
    acc = bls_ref[0] + bls_ref[1] + bls_ref[2] + bls_ref[3]
    for i, h_ref in enumerate((h1_ref, h2_ref, h3_ref, h4_ref)):
        hp = h_ref[...]
        sums = (jnp.dot(jnp.dot(ae, hp, precision=_PREC), se_ref[...],
                        precision=_PREC)
                + jnp.dot(jnp.dot(ao, hp, precision=_PREC), so_ref[...],
                          precision=_PREC))
        acc = acc + jnp.dot(sums * inv_cnt, wlts_ref[i], precision=_PREC)
    out_ref[...] = acc


_H_OUT = jax.ShapeDtypeStruct((_NP, _PACK), jnp.float32)

_first_call = pl.pallas_call(_first_body, out_shape=_H_OUT)
_layer_call = pl.pallas_call(_layer_body, out_shape=_H_OUT)
_pool_call = pl.pallas_call(
    _pool_body,
    out_shape=jax.ShapeDtypeStruct((N_GRAPHS, OUT_DIM), jnp.float32))


def _blockdiag(w):
    z = jnp.zeros_like(w)
    return jnp.block([[w, z], [z, w]])


def _tile2(v):
    return jnp.concatenate([v, v]).reshape(1, -1)


def _mlp_args(p):
    return (_blockdiag(p["W1"].T), _tile2(p["b1"]), _tile2(p["g1"]),
            _tile2(p["be1"]), _blockdiag(p["W2"].T), _tile2(p["b2"]),
            _tile2(p["g2"]), _tile2(p["be2"]))


def kernel(x, edge_index, batch, params):
    edges3d = edge_index.reshape(2, _NCHUNKS, _CHUNK)
    zrows = jnp.zeros((_ROWS_PER_TILE, HID_DIM), jnp.float32)

    eye = jnp.eye(HID_DIM, dtype=jnp.float32)
    zed = jnp.zeros((HID_DIM, HID_DIM), jnp.float32)
    fold = jnp.concatenate([eye, eye], axis=0)
    unfold = jnp.concatenate([eye, eye], axis=1)
    se = jnp.concatenate([eye, zed], axis=0)
    so = jnp.concatenate([zed, eye], axis=0)

    x_p = x.reshape(_NP, 2 * IN_DIM)
    b2d = batch.reshape(_NP, 2)
    b_even = b2d[:, 0].reshape(1, _NP)
    b_odd = b2d[:, 1].reshape(1, _NP)

    lin = params["lin"]
    hs = [_first_call(x_p, fold, unfold, *_mlp_args(params["first_h"]))]
    for layer in range(1, LAYERS):
        agg = _sc_edge_agg(hs[-1].reshape(N_NODES, HID_DIM), edges3d, zrows)
        agg_p = agg.reshape(_NC, _N_PAD // 2, _PACK)
        eps = params["eps"][layer - 1].reshape(1, 1)
        hs.append(_layer_call(hs[-1], agg_p, eps, fold, unfold,
                              *_mlp_args(params["nns"][layer - 1])))

    wlts = jnp.stack([lin[i]["W"].T for i in range(LAYERS)])
    bls = jnp.stack([lin[i]["b"].reshape(1, -1) for i in range(LAYERS)])
    return _pool_call(b_even, b_odd, se, so, *hs, wlts, bls)


# speedup vs baseline: 16.8059x; 1.0139x over previous
import functools

import jax
import jax.numpy as jnp
from jax import lax
from jax.experimental import pallas as pl
from jax.experimental.pallas import tpu as pltpu
from jax.experimental.pallas import tpu_sc as plsc

N_NODES = 10000
N_EDGES = 320000
N_GRAPHS = 64
IN_DIM = 128
HID_DIM = 64
OUT_DIM = 64
LAYERS = 4
EPS_BN = 1e-5


_NC = 2
_NS = 16
_NW = _NC * _NS
_CHUNK = 125
_NCHUNKS = N_EDGES // _CHUNK
_CH_PER_W = _NCHUNKS // _NW
_ROWS_PER_TILE = 632
_N_PAD = _ROWS_PER_TILE * _NS


_DEPTH = 8


def _sc_edge_agg(h, edges3d, zrows):
    mesh = plsc.VectorSubcoreMesh(core_axis_name="c", subcore_axis_name="s")

    @functools.partial(
        pl.kernel,
        mesh=mesh,
        out_type=jax.ShapeDtypeStruct((_NC, _N_PAD, HID_DIM), jnp.float32),
        scratch_types=[
            pltpu.VMEM((_CH_PER_W, _CHUNK), jnp.int32),
            pltpu.VMEM((_CH_PER_W, _CHUNK), jnp.int32),
            [pltpu.VMEM((_CHUNK, HID_DIM), jnp.float32) for _ in range(_DEPTH)],
            [pltpu.SemaphoreType.DMA for _ in range(_DEPTH)],
            [pltpu.SemaphoreType.DMA for _ in range(_DEPTH)],
            pltpu.VMEM_SHARED((_N_PAD, HID_DIM), jnp.float32),
        ],
        compiler_params=pltpu.CompilerParams(use_tc_tiling_on_sc=False),
    )
    def agg_kernel(h_hbm, e_hbm, z_hbm, out_hbm,
                   src_v, dst_v, rows, gsems, ssems, acc_sh):
        c = lax.axis_index("c")
        s = lax.axis_index("s")
        w = s * _NC + c

        r0 = s * _ROWS_PER_TILE
        zslab = acc_sh.at[pl.ds(r0, _ROWS_PER_TILE)]
        pltpu.async_copy(z_hbm, zslab, ssems[_DEPTH - 1])
        pltpu.sync_copy(e_hbm.at[0, pl.ds(w * _CH_PER_W, _CH_PER_W)], src_v)
        pltpu.sync_copy(e_hbm.at[1, pl.ds(w * _CH_PER_W, _CH_PER_W)], dst_v)

        def gstart(chunk, p):
            pltpu.async_copy(h_hbm.at[src_v.at[chunk]], rows[p], gsems[p])

        def gwait(chunk, p):
            pltpu.make_async_copy(h_hbm.at[src_v.at[chunk]], rows[p],
                                  gsems[p]).wait()

        def sstart(chunk, p):
            pltpu.async_copy(rows[p], acc_sh.at[dst_v.at[chunk]], ssems[p],
                             add=True)

        def swait(chunk, p):
            pltpu.make_async_copy(rows[p], acc_sh.at[dst_v.at[chunk]],
                                  ssems[p]).wait()

        for cc in range(_DEPTH):
            gstart(cc, cc)
        pltpu.make_async_copy(z_hbm, zslab, ssems[_DEPTH - 1]).wait()
        plsc.subcore_barrier()
        for cc in range(_DEPTH - 2):
            gwait(cc, cc)
            sstart(cc, cc)

        def body(j, carry):
            c0 = (_DEPTH - 2) + _DEPTH * j
            for k in range(_DEPTH):
                ck = c0 + k
                p = (_DEPTH - 2 + k) % _DEPTH
                q = k
                swait(ck - (_DEPTH - 2), q)
                gstart(ck + 2, q)
                gwait(ck, p)
                sstart(ck, p)
            return carry

        n_steady = _CH_PER_W - _DEPTH
        lax.fori_loop(0, n_steady // _DEPTH, body, 0)
        for ck in range(_CH_PER_W - 2, _CH_PER_W):
            gwait(ck, ck % _DEPTH)
            sstart(ck, ck % _DEPTH)
        for ck in range(_CH_PER_W - _DEPTH, _CH_PER_W):
            swait(ck, ck % _DEPTH)

        plsc.subcore_barrier()
        pltpu.sync_copy(acc_sh.at[pl.ds(r0, _ROWS_PER_TILE)],
                        out_hbm.at[c, pl.ds(r0, _ROWS_PER_TILE)])

    return agg_kernel(h, edges3d, zrows)


_PREC = lax.Precision.DEFAULT


_NP = N_NODES // 2
_PACK = 2 * HID_DIM


_HI = lax.Precision.HIGHEST


def _bn_relu_packed(y, fold_ref, unfold_ref, g2, be2):
    m = jnp.dot(jnp.mean(y, axis=0, keepdims=True), fold_ref[...],
                precision=_HI) * 0.5
    yc = y - jnp.dot(m, unfold_ref[...], precision=_HI)
    v = jnp.dot(jnp.mean(yc * yc, axis=0, keepdims=True), fold_ref[...],
                precision=_HI) * 0.5
    vb = jnp.dot(v, unfold_ref[...], precision=_HI)
    return jnp.maximum(yc * lax.rsqrt(vb + EPS_BN) * g2 + be2, 0.0)


def _first_body(x_ref, fold_ref, unfold_ref, w1t_ref, b1_ref, g1_ref,
                be1_ref, w2t_ref, b2_ref, g2_ref, be2_ref, h_ref):
    y = jnp.dot(x_ref[...], w1t_ref[...], precision=_PREC) + b1_ref[...]
    y = _bn_relu_packed(y, fold_ref, unfold_ref, g1_ref[...], be1_ref[...])
    y = jnp.dot(y, w2t_ref[...], precision=_PREC) + b2_ref[...]
    h_ref[...] = _bn_relu_packed(y, fold_ref, unfold_ref, g2_ref[...],
                                 be2_ref[...])


def _layer_body(h_in_ref, agg_ref, eps_ref, fold_ref, unfold_ref, w1t_ref,
                b1_ref, g1_ref, be1_ref, w2t_ref, b2_ref, g2_ref, be2_ref,
                h_ref):
    u = (h_in_ref[...] * (1.0 + eps_ref[...])
         + agg_ref[0, :_NP, :] + agg_ref[1, :_NP, :])
    y = jnp.dot(u, w1t_ref[...], precision=_PREC) + b1_ref[...]
    y = _bn_relu_packed(y, fold_ref, unfold_ref, g1_ref[...], be1_ref[...])
    y = jnp.dot(y, w2t_ref[...], precision=_PREC) + b2_ref[...]
    h_ref[...] = _bn_relu_packed(y, fold_ref, unfold_ref, g2_ref[...],
                                 be2_ref[...])


def _onehots(be_ref, bo_ref):
    ae = (lax.broadcasted_iota(jnp.int32, (N_GRAPHS, _NP), 0)
          == be_ref[...]).astype(jnp.float32)
    ao = (lax.broadcasted_iota(jnp.int32, (N_GRAPHS, _NP), 0)
          == bo_ref[...]).astype(jnp.float32)
    cnt = (jnp.sum(ae, axis=1, keepdims=True)
           + jnp.sum(ao, axis=1, keepdims=True))
    return ae, ao, 1.0 / jnp.maximum(cnt, 1.0)


def _seg_mean(ae, ao, inv_cnt, hp, se_ref, so_ref):
    sums = (jnp.dot(jnp.dot(ae, hp, precision=_PREC), se_ref[...],
                    precision=_PREC)
            + jnp.dot(jnp.dot(ao, hp, precision=_PREC), so_ref[...],
                      precision=_PREC))
    return sums * inv_cnt


def _poolpre_body(be_ref, bo_ref, se_ref, so_ref, h1_ref, h2_ref, h3_ref,
                  wlts_ref, blsum_ref, out_ref):
    ae, ao, inv_cnt = _onehots(be_ref, bo_ref)
    acc = blsum_ref[...]
    for i, h_ref in enumerate((h1_ref, h2_ref, h3_ref)):
        pooled = _seg_mean(ae, ao, inv_cnt, h_ref[...], se_ref, so_ref)
        acc = acc + jnp.dot(pooled, wlts_ref[i], precision=_PREC)
    out_ref[...] = acc


def _poolfin_body(be_ref, bo_ref, se_ref, so_ref, h4_ref, w4t_ref, pacc_ref,
                  out_ref):
    ae, ao, inv_cnt = _onehots(be_ref, bo_ref)
    pooled = _seg_mean(ae, ao, inv_cnt, h4_ref[...], se_ref, so_ref)
    out_ref[...] = pacc_ref[...] + jnp.dot(pooled, w4t_ref[...],
                                           precision=_PREC)


_H_OUT = jax.ShapeDtypeStruct((_NP, _PACK), jnp.float32)
_OUT_SD = jax.ShapeDtypeStruct((N_GRAPHS, OUT_DIM), jnp.float32)

_first_call = pl.pallas_call(_first_body, out_shape=_H_OUT)
_layer_call = pl.pallas_call(_layer_body, out_shape=_H_OUT)
_poolpre_call = pl.pallas_call(_poolpre_body, out_shape=_OUT_SD)
_poolfin_call = pl.pallas_call(_poolfin_body, out_shape=_OUT_SD)


def _blockdiag(w):
    z = jnp.zeros_like(w)
    return jnp.block([[w, z], [z, w]])


def _tile2(v):
    return jnp.concatenate([v, v]).reshape(1, -1)


def _mlp_args(p):
    return (_blockdiag(p["W1"].T), _tile2(p["b1"]), _tile2(p["g1"]),
            _tile2(p["be1"]), _blockdiag(p["W2"].T), _tile2(p["b2"]),
            _tile2(p["g2"]), _tile2(p["be2"]))


def kernel(x, edge_index, batch, params):
    edges3d = edge_index.reshape(2, _NCHUNKS, _CHUNK)
    zrows = jnp.zeros((_ROWS_PER_TILE, HID_DIM), jnp.float32)

    eye = jnp.eye(HID_DIM, dtype=jnp.float32)
    zed = jnp.zeros((HID_DIM, HID_DIM), jnp.float32)
    fold = jnp.concatenate([eye, eye], axis=0)
    unfold = jnp.concatenate([eye, eye], axis=1)
    se = jnp.concatenate([eye, zed], axis=0)
    so = jnp.concatenate([zed, eye], axis=0)

    x_p = x.reshape(_NP, 2 * IN_DIM)
    b2d = batch.reshape(_NP, 2)
    b_even = b2d[:, 0].reshape(1, _NP)
    b_odd = b2d[:, 1].reshape(1, _NP)

    lin = params["lin"]
    hs = [_first_call(x_p, fold, unfold, *_mlp_args(params["first_h"]))]
    for layer in range(1, LAYERS):
        agg = _sc_edge_agg(hs[-1].reshape(N_NODES, HID_DIM), edges3d, zrows)
        agg_p = agg.reshape(_NC, _N_PAD // 2, _PACK)
        eps = params["eps"][layer - 1].reshape(1, 1)
        hs.append(_layer_call(hs[-1], agg_p, eps, fold, unfold,
                              *_mlp_args(params["nns"][layer - 1])))

    wlts3 = jnp.stack([lin[i]["W"].T for i in range(LAYERS - 1)])
    blsum = sum(lin[i]["b"] for i in range(LAYERS)).reshape(1, -1)
    pacc = _poolpre_call(b_even, b_odd, se, so, hs[0], hs[1], hs[2],
                         wlts3, blsum)
    return _poolfin_call(b_even, b_odd, se, so, hs[3], lin[3]["W"].T, pacc)


# speedup vs baseline: 17.7272x; 1.0548x over previous
import functools

import jax
import jax.numpy as jnp
from jax import lax
from jax.experimental import pallas as pl
from jax.experimental.pallas import tpu as pltpu
from jax.experimental.pallas import tpu_sc as plsc

N_NODES = 10000
N_EDGES = 320000
N_GRAPHS = 64
IN_DIM = 128
HID_DIM = 64
OUT_DIM = 64
LAYERS = 4
EPS_BN = 1e-5


_NC = 2
_NS = 16
_NW = _NC * _NS
_CHUNK = 125
_NCHUNKS = N_EDGES // _CHUNK
_CH_PER_W = _NCHUNKS // _NW
_ROWS_PER_TILE = 632
_N_PAD = _ROWS_PER_TILE * _NS


_DEPTH = 8


def _sc_edge_agg(h, edges3d, zrows):
    mesh = plsc.VectorSubcoreMesh(core_axis_name="c", subcore_axis_name="s")

    @functools.partial(
        pl.kernel,
        mesh=mesh,
        out_type=jax.ShapeDtypeStruct((_NC, _N_PAD, HID_DIM), jnp.float32),
        scratch_types=[
            pltpu.VMEM((_CH_PER_W, _CHUNK), jnp.int32),
            pltpu.VMEM((_CH_PER_W, _CHUNK), jnp.int32),
            [pltpu.VMEM((_CHUNK, HID_DIM), jnp.float32) for _ in range(_DEPTH)],
            [pltpu.SemaphoreType.DMA for _ in range(_DEPTH)],
            [pltpu.SemaphoreType.DMA for _ in range(_DEPTH)],
            pltpu.VMEM_SHARED((_N_PAD, HID_DIM), jnp.float32),
        ],
        compiler_params=pltpu.CompilerParams(use_tc_tiling_on_sc=False),
    )
    def agg_kernel(h_hbm, e_hbm, z_hbm, out_hbm,
                   src_v, dst_v, rows, gsems, ssems, acc_sh):
        c = lax.axis_index("c")
        s = lax.axis_index("s")
        w = s * _NC + c

        r0 = s * _ROWS_PER_TILE
        zslab = acc_sh.at[pl.ds(r0, _ROWS_PER_TILE)]
        pltpu.async_copy(z_hbm, zslab, ssems[_DEPTH - 1])
        pltpu.sync_copy(e_hbm.at[0, pl.ds(w * _CH_PER_W, _CH_PER_W)], src_v)
        pltpu.sync_copy(e_hbm.at[1, pl.ds(w * _CH_PER_W, _CH_PER_W)], dst_v)

        def gstart(chunk, p):
            pltpu.async_copy(h_hbm.at[src_v.at[chunk]], rows[p], gsems[p])

        def gwait(chunk, p):
            pltpu.make_async_copy(h_hbm.at[src_v.at[chunk]], rows[p],
                                  gsems[p]).wait()

        def sstart(chunk, p):
            pltpu.async_copy(rows[p], acc_sh.at[dst_v.at[chunk]], ssems[p],
                             add=True)

        def swait(chunk, p):
            pltpu.make_async_copy(rows[p], acc_sh.at[dst_v.at[chunk]],
                                  ssems[p]).wait()

        for cc in range(_DEPTH):
            gstart(cc, cc)
        pltpu.make_async_copy(z_hbm, zslab, ssems[_DEPTH - 1]).wait()
        plsc.subcore_barrier()
        _AHEAD = _DEPTH // 2
        for cc in range(_AHEAD):
            gwait(cc, cc)
            sstart(cc, cc)

        def body(j, carry):
            c0 = _AHEAD + _DEPTH * j
            for k in range(_DEPTH):
                ck = c0 + k
                p = (_AHEAD + k) % _DEPTH
                q = k
                swait(ck - _AHEAD, q)
                gstart(ck + _AHEAD, q)
                gwait(ck, p)
                sstart(ck, p)
            return carry

        n_steady = _CH_PER_W - 2 * _AHEAD
        lax.fori_loop(0, n_steady // _DEPTH, body, 0)
        for ck in range(_CH_PER_W - _AHEAD, _CH_PER_W):
            gwait(ck, ck % _DEPTH)
            sstart(ck, ck % _DEPTH)
        for ck in range(_CH_PER_W - _DEPTH, _CH_PER_W):
            swait(ck, ck % _DEPTH)

        plsc.subcore_barrier()
        pltpu.sync_copy(acc_sh.at[pl.ds(r0, _ROWS_PER_TILE)],
                        out_hbm.at[c, pl.ds(r0, _ROWS_PER_TILE)])

    return agg_kernel(h, edges3d, zrows)


_PREC = lax.Precision.DEFAULT


_NP = N_NODES // 2
_PACK = 2 * HID_DIM


_HI = lax.Precision.HIGHEST


def _bn_relu_packed(y, fold_ref, unfold_ref, g2, be2):
    m = jnp.dot(jnp.mean(y, axis=0, keepdims=True), fold_ref[...],
                precision=_HI) * 0.5
    yc = y - jnp.dot(m, unfold_ref[...], precision=_HI)
    v = jnp.dot(jnp.mean(yc * yc, axis=0, keepdims=True), fold_ref[...],
                precision=_HI) * 0.5
    vb = jnp.dot(v, unfold_ref[...], precision=_HI)
    return jnp.maximum(yc * lax.rsqrt(vb + EPS_BN) * g2 + be2, 0.0)


def _first_body(x_ref, fold_ref, unfold_ref, w1t_ref, b1_ref, g1_ref,
                be1_ref, w2t_ref, b2_ref, g2_ref, be2_ref, h_ref):
    y = jnp.dot(x_ref[...], w1t_ref[...], precision=_PREC) + b1_ref[...]
    y = _bn_relu_packed(y, fold_ref, unfold_ref, g1_ref[...], be1_ref[...])
    y = jnp.dot(y, w2t_ref[...], precision=_PREC) + b2_ref[...]
    h_ref[...] = _bn_relu_packed(y, fold_ref, unfold_ref, g2_ref[...],
                                 be2_ref[...])


def _layer_body(h_in_ref, agg_ref, eps_ref, fold_ref, unfold_ref, w1t_ref,
                b1_ref, g1_ref, be1_ref, w2t_ref, b2_ref, g2_ref, be2_ref,
                h_ref):
    u = (h_in_ref[...] * (1.0 + eps_ref[...])
         + agg_ref[0, :_NP, :] + agg_ref[1, :_NP, :])
    y = jnp.dot(u, w1t_ref[...], precision=_PREC) + b1_ref[...]
    y = _bn_relu_packed(y, fold_ref, unfold_ref, g1_ref[...], be1_ref[...])
    y = jnp.dot(y, w2t_ref[...], precision=_PREC) + b2_ref[...]
    h_ref[...] = _bn_relu_packed(y, fold_ref, unfold_ref, g2_ref[...],
                                 be2_ref[...])


def _onehots(be_ref, bo_ref):
    ae = (lax.broadcasted_iota(jnp.int32, (N_GRAPHS, _NP), 0)
          == be_ref[...]).astype(jnp.float32)
    ao = (lax.broadcasted_iota(jnp.int32, (N_GRAPHS, _NP), 0)
          == bo_ref[...]).astype(jnp.float32)
    cnt = (jnp.sum(ae, axis=1, keepdims=True)
           + jnp.sum(ao, axis=1, keepdims=True))
    return ae, ao, 1.0 / jnp.maximum(cnt, 1.0)


def _seg_mean(ae, ao, inv_cnt, hp, se_ref, so_ref):
    sums = (jnp.dot(jnp.dot(ae, hp, precision=_PREC), se_ref[...],
                    precision=_PREC)
            + jnp.dot(jnp.dot(ao, hp, precision=_PREC), so_ref[...],
                      precision=_PREC))
    return sums * inv_cnt


def _poolpre_body(be_ref, bo_ref, se_ref, so_ref, h1_ref, h2_ref, h3_ref,
                  wlts_ref, blsum_ref, out_ref):
    ae, ao, inv_cnt = _onehots(be_ref, bo_ref)
    acc = blsum_ref[...]
    for i, h_ref in enumerate((h1_ref, h2_ref, h3_ref)):
        pooled = _seg_mean(ae, ao, inv_cnt, h_ref[...], se_ref, so_ref)
        acc = acc + jnp.dot(pooled, wlts_ref[i], precision=_PREC)
    out_ref[...] = acc


def _poolfin_body(be_ref, bo_ref, se_ref, so_ref, h4_ref, w4t_ref, pacc_ref,
                  out_ref):
    ae, ao, inv_cnt = _onehots(be_ref, bo_ref)
    pooled = _seg_mean(ae, ao, inv_cnt, h4_ref[...], se_ref, so_ref)
    out_ref[...] = pacc_ref[...] + jnp.dot(pooled, w4t_ref[...],
                                           precision=_PREC)


_H_OUT = jax.ShapeDtypeStruct((_NP, _PACK), jnp.float32)
_OUT_SD = jax.ShapeDtypeStruct((N_GRAPHS, OUT_DIM), jnp.float32)

_first_call = pl.pallas_call(_first_body, out_shape=_H_OUT)
_layer_call = pl.pallas_call(_layer_body, out_shape=_H_OUT)
_poolpre_call = pl.pallas_call(_poolpre_body, out_shape=_OUT_SD)
_poolfin_call = pl.pallas_call(_poolfin_body, out_shape=_OUT_SD)


def _blockdiag(w):
    z = jnp.zeros_like(w)
    return jnp.block([[w, z], [z, w]])


def _tile2(v):
    return jnp.concatenate([v, v]).reshape(1, -1)


def _mlp_args(p):
    return (_blockdiag(p["W1"].T), _tile2(p["b1"]), _tile2(p["g1"]),
            _tile2(p["be1"]), _blockdiag(p["W2"].T), _tile2(p["b2"]),
            _tile2(p["g2"]), _tile2(p["be2"]))


def kernel(x, edge_index, batch, params):
    edges3d = edge_index.reshape(2, _NCHUNKS, _CHUNK)
    zrows = jnp.zeros((_ROWS_PER_TILE, HID_DIM), jnp.float32)

    eye = jnp.eye(HID_DIM, dtype=jnp.float32)
    zed = jnp.zeros((HID_DIM, HID_DIM), jnp.float32)
    fold = jnp.concatenate([eye, eye], axis=0)
    unfold = jnp.concatenate([eye, eye], axis=1)
    se = jnp.concatenate([eye, zed], axis=0)
    so = jnp.concatenate([zed, eye], axis=0)

    x_p = x.reshape(_NP, 2 * IN_DIM)
    b2d = batch.reshape(_NP, 2)
    b_even = b2d[:, 0].reshape(1, _NP)
    b_odd = b2d[:, 1].reshape(1, _NP)

    lin = params["lin"]
    hs = [_first_call(x_p, fold, unfold, *_mlp_args(params["first_h"]))]
    for layer in range(1, LAYERS):
        agg = _sc_edge_agg(hs[-1].reshape(N_NODES, HID_DIM), edges3d, zrows)
        agg_p = agg.reshape(_NC, _N_PAD // 2, _PACK)
        eps = params["eps"][layer - 1].reshape(1, 1)
        hs.append(_layer_call(hs[-1], agg_p, eps, fold, unfold,
                              *_mlp_args(params["nns"][layer - 1])))

    wlts3 = jnp.stack([lin[i]["W"].T for i in range(LAYERS - 1)])
    blsum = sum(lin[i]["b"] for i in range(LAYERS)).reshape(1, -1)
    pacc = _poolpre_call(b_even, b_odd, se, so, hs[0], hs[1], hs[2],
                         wlts3, blsum)
    return _poolfin_call(b_even, b_odd, se, so, hs[3], lin[3]["W"].T, pacc)
